# Initial kernel scaffold; baseline (speedup 1.0000x reference)
import jax
import jax.numpy as jnp
from jax.experimental import pallas as pl


def kernel(proteinEmbedding, nodeFeatures, edgeIndex, edgeFeatures, params):
    raise NotImplementedError("write your pallas kernel here")


# speedup vs baseline: 2.9885x; 2.9885x over previous
import functools

import jax
import jax.numpy as jnp
from jax import lax
from jax.experimental import pallas as pl
from jax.experimental.pallas import tpu as pltpu
from jax.experimental.pallas import tpu_sc as plsc

N_NODES = 50000
N_EDGES = 800000

NCORE = 2
NSUB = 16
NW = NCORE * NSUB
BATCH = 128
ROWS = 200
KSUB = 40
EPAD = NW * ROWS * BATCH
ACC_ROWS = 50176
ZROWS = 112
FLUSH = 3136

_MESH = plsc.VectorSubcoreMesh(core_axis_name="c", subcore_axis_name="s")
_SC_PARAMS = pltpu.CompilerParams(use_tc_tiling_on_sc=False)


def _zero_zbuf(zbuf):
    def body(i, _):
        zbuf[i // 2, pl.ds((i % 2) * 16, 16)] = jnp.zeros((16,), jnp.float32)
        return 0
    lax.fori_loop(0, ZROWS * 2, body, 0, unroll=4)


def _zero_stripe(acc, zbuf, s):
    for z in range(FLUSH // ZROWS):
        pltpu.sync_copy(zbuf, acc.at[pl.ds(s * FLUSH + z * ZROWS, ZROWS)])


@functools.partial(
    pl.kernel,
    out_type=jax.ShapeDtypeStruct((4, ACC_ROWS, 32), jnp.float32),
    mesh=_MESH,
    compiler_params=_SC_PARAMS,
    scratch_types=[
        pltpu.VMEM_SHARED((ACC_ROWS, 32), jnp.float32),
        pltpu.VMEM((KSUB, BATCH), jnp.int32),
        pltpu.VMEM((KSUB, BATCH), jnp.int32),
        pltpu.VMEM((2, BATCH, 32), jnp.float32),
        pltpu.VMEM((ZROWS, 32), jnp.float32),
        pltpu.SemaphoreType.DMA,
        pltpu.SemaphoreType.DMA,
    ],
)
def _sc_seg4(src_hbm, dst_hbm, x_hbm, out_hbm,
             acc, srcv, dstv, rows, zbuf, sem_g, sem_s):
    c = lax.axis_index("c")
    s = lax.axis_index("s")
    _zero_zbuf(zbuf)
    for ci in range(2):
        chunk = c * 2 + ci
        _zero_stripe(acc, zbuf, s)
        plsc.subcore_barrier()
        for w in range(2):
            wid = s * 2 + w

            def kbody(k, _):
                pltpu.sync_copy(src_hbm.at[wid].at[pl.ds(k * KSUB, KSUB)], srcv)
                pltpu.sync_copy(dst_hbm.at[wid].at[pl.ds(k * KSUB, KSUB)], dstv)

                def body(j, _):
                    g = pltpu.async_copy(
                        x_hbm.at[chunk].at[srcv.at[j]], rows.at[j % 2], sem_g)
                    g.wait()
                    pltpu.sync_copy(rows.at[j % 2], acc.at[dstv.at[j]], add=True)
                    return 0
                lax.fori_loop(0, KSUB, body, 0)
                return 0
            lax.fori_loop(0, ROWS // KSUB, kbody, 0)
        plsc.subcore_barrier()
        pltpu.sync_copy(acc.at[pl.ds(s * FLUSH, FLUSH)],
                        out_hbm.at[chunk].at[pl.ds(s * FLUSH, FLUSH)])
        plsc.subcore_barrier()


@functools.partial(
    pl.kernel,
    out_type=jax.ShapeDtypeStruct((2, ACC_ROWS, 32), jnp.float32),
    mesh=_MESH,
    compiler_params=_SC_PARAMS,
    scratch_types=[
        pltpu.VMEM_SHARED((ACC_ROWS, 32), jnp.float32),
        pltpu.VMEM((KSUB, BATCH), jnp.int32),
        pltpu.VMEM((KSUB, BATCH), jnp.int32),
        pltpu.VMEM((2, BATCH, 32), jnp.float32),
        pltpu.VMEM((ZROWS, 32), jnp.float32),
        pltpu.SemaphoreType.DMA,
        pltpu.SemaphoreType.DMA,
    ],
)
def _sc_seg1(src_hbm, dst_hbm, x_hbm, out_hbm,
             acc, srcv, dstv, rows, zbuf, sem_g, sem_s):
    c = lax.axis_index("c")
    s = lax.axis_index("s")
    _zero_zbuf(zbuf)
    _zero_stripe(acc, zbuf, s)
    plsc.subcore_barrier()
    wid = s * 2 + c

    def kbody(k, _):
        pltpu.sync_copy(src_hbm.at[wid].at[pl.ds(k * KSUB, KSUB)], srcv)
        pltpu.sync_copy(dst_hbm.at[wid].at[pl.ds(k * KSUB, KSUB)], dstv)

        def body(j, _):
            g = pltpu.async_copy(x_hbm.at[srcv.at[j]], rows.at[j % 2], sem_g)
            g.wait()
            pltpu.sync_copy(rows.at[j % 2], acc.at[dstv.at[j]], add=True)
            return 0
        lax.fori_loop(0, KSUB, body, 0)
        return 0
    lax.fori_loop(0, ROWS // KSUB, kbody, 0)
    plsc.subcore_barrier()
    pltpu.sync_copy(acc.at[pl.ds(s * FLUSH, FLUSH)],
                    out_hbm.at[c].at[pl.ds(s * FLUSH, FLUSH)])


@functools.partial(
    pl.kernel,
    out_type=jax.ShapeDtypeStruct((2, ACC_ROWS, 8), jnp.float32),
    mesh=_MESH,
    compiler_params=_SC_PARAMS,
    scratch_types=[
        pltpu.VMEM_SHARED((ACC_ROWS, 8), jnp.float32),
        pltpu.VMEM((KSUB, BATCH), jnp.int32),
        pltpu.VMEM((BATCH, 8), jnp.float32),
        pltpu.SemaphoreType.DMA,
    ],
)
def _sc_efagg(dst_hbm, efx_hbm, z8_hbm, out_hbm, acc, dstv, rowbuf, sem):
    c = lax.axis_index("c")
    s = lax.axis_index("s")
    pltpu.sync_copy(z8_hbm, acc.at[pl.ds(s * FLUSH, FLUSH)])
    plsc.subcore_barrier()
    wid = s * 2 + c
    base = wid * (ROWS * BATCH)

    def kbody(k, _):
        pltpu.sync_copy(dst_hbm.at[wid].at[pl.ds(k * KSUB, KSUB)], dstv)

        def body(j, _):
            pltpu.sync_copy(
                efx_hbm.at[pl.ds(base + (k * KSUB + j) * BATCH, BATCH)], rowbuf)
            pltpu.sync_copy(rowbuf, acc.at[dstv.at[j]], add=True)
            return 0
        lax.fori_loop(0, KSUB, body, 0)
        return 0
    lax.fori_loop(0, ROWS // KSUB, kbody, 0)
    plsc.subcore_barrier()
    pltpu.sync_copy(acc.at[pl.ds(s * FLUSH, FLUSH)],
                    out_hbm.at[c].at[pl.ds(s * FLUSH, FLUSH)])


BN = 1000
NBLK = 50


def _tc_layer_body(ns, with_res, *refs):
    if with_res:
        s_ref, ws_ref, ef_ref, e8_ref, gb_ref, r_ref, out_ref = refs
    else:
        s_ref, ws_ref, ef_ref, e8_ref, gb_ref, out_ref = refs
        r_ref = None
    agg = jnp.dot(s_ref[0], ws_ref[0], preferred_element_type=jnp.float32)
    for n in range(1, ns):
        agg += jnp.dot(s_ref[n], ws_ref[n], preferred_element_type=jnp.float32)
    agg += jnp.dot(ef_ref[0] + ef_ref[1], e8_ref[...],
                   preferred_element_type=jnp.float32)
    h = jnp.maximum(agg, 0.0)
    mu = jnp.mean(h, axis=1, keepdims=True)
    var = jnp.mean((h - mu) ** 2, axis=1, keepdims=True)
    h = (h - mu) / jnp.sqrt(var + 1e-5) * gb_ref[0:1] + gb_ref[1:2]
    for ch in range(4):
        piece = h[:, ch * 32:(ch + 1) * 32]
        if r_ref is not None:
            piece = piece + r_ref[ch]
        out_ref[ch] = piece


def _make_tc_layer(ns, with_res):
    in_specs = [
        pl.BlockSpec((ns, BN, 32), lambda i: (0, i, 0)),
        pl.BlockSpec((ns, 32, 128), lambda i: (0, 0, 0)),
        pl.BlockSpec((2, BN, 8), lambda i: (0, i, 0)),
        pl.BlockSpec((8, 128), lambda i: (0, 0)),
        pl.BlockSpec((2, 128), lambda i: (0, 0)),
    ]
    if with_res:
        in_specs.append(pl.BlockSpec((4, BN, 32), lambda i: (0, i, 0)))
    return pl.pallas_call(
        functools.partial(_tc_layer_body, ns, with_res),
        grid=(NBLK,),
        in_specs=in_specs,
        out_specs=pl.BlockSpec((4, BN, 32), lambda i: (0, i, 0)),
        out_shape=jax.ShapeDtypeStruct((4, N_NODES, 32), jnp.float32),
    )


_tc_layer0 = _make_tc_layer(2, False)
_tc_layer1 = _make_tc_layer(4, True)
_tc_layer2 = _make_tc_layer(4, False)


def _ln_row(h, g, b):
    mu = jnp.mean(h, axis=1, keepdims=True)
    var = jnp.mean((h - mu) ** 2, axis=1, keepdims=True)
    return (h - mu) / jnp.sqrt(var + 1e-5) * g + b


def _attn_head_body(x_ref, pe_ref, wq_ref, bq_ref, wk_ref, bk_ref,
                    wv_ref, bv_ref, wo_ref, bo_ref, g_ref, hm_ref,
                    w1a_ref, w1b_ref, b1_ref, g1_ref, n1_ref,
                    w2_ref, b2_ref, g2_ref, n2_ref,
                    w3_ref, b3_ref, g3_ref, n3_ref,
                    w4_ref, b4_ref, out_ref,
                    qs, m_sc, l_sc, acc_sc):
    i = pl.program_id(0)

    @pl.when(i == 0)
    def _init():
        q = jnp.dot(pe_ref[...], wq_ref[...],
                    preferred_element_type=jnp.float32) + bq_ref[...]
        qs[...] = q / jnp.sqrt(32.0)
        m_sc[...] = jnp.full((4, 1), -1e30, jnp.float32)
        l_sc[...] = jnp.zeros((4, 1), jnp.float32)
        acc_sc[...] = jnp.zeros((4, 128), jnp.float32)

    k = jnp.dot(x_ref[0], wk_ref[0:32], preferred_element_type=jnp.float32)
    v = jnp.dot(x_ref[0], wv_ref[0:32], preferred_element_type=jnp.float32)
    for ch in range(1, 4):
        k += jnp.dot(x_ref[ch], wk_ref[ch * 32:(ch + 1) * 32],
                     preferred_element_type=jnp.float32)
        v += jnp.dot(x_ref[ch], wv_ref[ch * 32:(ch + 1) * 32],
                     preferred_element_type=jnp.float32)
    k = k + bk_ref[...]
    v = v + bv_ref[...]
    kq = k * qs[...]
    sT = lax.dot_general(g_ref[...], kq, (((0,), (1,)), ((), ())),
                         preferred_element_type=jnp.float32)
    m_blk = jnp.max(sT, axis=1, keepdims=True)
    m_new = jnp.maximum(m_sc[...], m_blk)
    alpha = jnp.exp(m_sc[...] - m_new)
    p = jnp.exp(sT - m_new)
    l_sc[...] = l_sc[...] * alpha + jnp.sum(p, axis=1, keepdims=True)
    acc_sc[...] = acc_sc[...] * alpha + lax.dot_general(
        p, v, (((1,), (0,)), ((), ())), preferred_element_type=jnp.float32)
    m_sc[...] = m_new

    @pl.when(i == NBLK - 1)
    def _final():
        att = acc_sc[...] / l_sc[...]
        att1 = jnp.sum(att * hm_ref[...], axis=0, keepdims=True)
        o = jnp.dot(att1, wo_ref[...],
                    preferred_element_type=jnp.float32) + bo_ref[...]
        h = (jnp.dot(pe_ref[...], w1a_ref[...], preferred_element_type=jnp.float32)
             + jnp.dot(o, w1b_ref[...], preferred_element_type=jnp.float32)
             + b1_ref[...])
        h = jnp.maximum(_ln_row(h, g1_ref[...], n1_ref[...]), 0.0)
        h = jnp.dot(h, w2_ref[...], preferred_element_type=jnp.float32) + b2_ref[...]
        h = jnp.maximum(_ln_row(h, g2_ref[...], n2_ref[...]), 0.0)
        h = jnp.dot(h, w3_ref[...], preferred_element_type=jnp.float32) + b3_ref[...]
        h = jnp.maximum(_ln_row(h, g3_ref[...], n3_ref[...]), 0.0)
        pred = jnp.dot(h, w4_ref[...], preferred_element_type=jnp.float32) + b4_ref[...]
        out_ref[...] = pred


def _full(shape):
    nd = len(shape)
    return pl.BlockSpec(shape, lambda i: (0,) * nd)


_attn_head = pl.pallas_call(
    _attn_head_body,
    grid=(NBLK,),
    in_specs=[
        pl.BlockSpec((4, BN, 32), lambda i: (0, i, 0)),
        _full((1, 480)), _full((480, 128)), _full((1, 128)),
        _full((128, 128)), _full((1, 128)),
        _full((128, 128)), _full((1, 128)),
        _full((128, 128)), _full((1, 128)),
        _full((128, 4)), _full((4, 128)),
        _full((480, 512)), _full((128, 512)), _full((1, 512)),
        _full((1, 512)), _full((1, 512)),
        _full((512, 256)), _full((1, 256)), _full((1, 256)), _full((1, 256)),
        _full((256, 128)), _full((1, 128)), _full((1, 128)), _full((1, 128)),
        _full((128, 1)), _full((1, 1)),
    ],
    out_specs=_full((1, 1)),
    out_shape=jax.ShapeDtypeStruct((1, 1), jnp.float32),
    scratch_shapes=[
        pltpu.VMEM((1, 128), jnp.float32),
        pltpu.VMEM((4, 1), jnp.float32),
        pltpu.VMEM((4, 1), jnp.float32),
        pltpu.VMEM((4, 128), jnp.float32),
    ],
)


def kernel(proteinEmbedding, nodeFeatures, edgeIndex, edgeFeatures, params):
    f32 = jnp.float32
    src = edgeIndex[:, 0]
    dst = edgeIndex[:, 1]
    pad = EPAD - N_EDGES
    srcp = jnp.concatenate([src, jnp.zeros((pad,), jnp.int32)]).reshape(NW, ROWS, BATCH)
    dstp = jnp.concatenate(
        [dst, jnp.full((pad,), N_NODES, jnp.int32)]).reshape(NW, ROWS, BATCH)
    efx = jnp.concatenate(
        [edgeFeatures, jnp.zeros((N_EDGES, 1), f32), jnp.ones((N_EDGES, 1), f32)], 1)
    efxp = jnp.concatenate([efx, jnp.zeros((pad, 8), f32)], 0)
    z8 = jnp.zeros((FLUSH, 8), f32)
    x0 = jnp.pad(nodeFeatures, ((0, 0), (0, 13)))

    gcn = params['gcn']
    ws = []
    e8s = []
    gbs = []
    for i, p in enumerate(gcn):
        if i == 0:
            w = jnp.pad(p['W'], ((0, 13), (0, 0)))
            ws.append(jnp.stack([w, w]))
        else:
            ws.append(p['W'].reshape(4, 32, 128))
        e8s.append(jnp.concatenate(
            [p['We'], jnp.zeros((1, 128), f32), (p['b'] + p['be'])[None]], 0))
        gbs.append(jnp.stack([p['g'], p['bn']]))

    EF = _sc_efagg(dstp, efxp, z8)
    S0 = _sc_seg1(srcp, dstp, x0)
    h0 = _tc_layer0(S0, ws[0], EF, e8s[0], gbs[0])
    S1 = _sc_seg4(srcp, dstp, h0)
    h1 = _tc_layer1(S1, ws[1], EF, e8s[1], gbs[1], h0)
    S2 = _sc_seg4(srcp, dstp, h1)
    h2 = _tc_layer2(S2, ws[2], EF, e8s[2], gbs[2])

    mlp = params['mlp']
    G = (jnp.arange(128)[:, None] // 32 == jnp.arange(4)[None, :]).astype(f32)
    HM = G.T
    pred = _attn_head(
        h2, proteinEmbedding[None], params['Wq'], params['bq'][None],
        params['Wk'], params['bk'][None], params['Wv'], params['bv'][None],
        params['Wo'], params['bo'][None], G, HM,
        mlp[0]['W'][:480], mlp[0]['W'][480:], mlp[0]['b'][None],
        mlp[0]['g'][None], mlp[0]['bn'][None],
        mlp[1]['W'], mlp[1]['b'][None], mlp[1]['g'][None], mlp[1]['bn'][None],
        mlp[2]['W'], mlp[2]['b'][None], mlp[2]['g'][None], mlp[2]['bn'][None],
        mlp[3]['W'], mlp[3]['b'][None],
    )
    return pred.reshape(1)


# speedup vs baseline: 3.6262x; 1.2134x over previous
import functools

import jax
import jax.numpy as jnp
from jax import lax
from jax.experimental import pallas as pl
from jax.experimental.pallas import tpu as pltpu
from jax.experimental.pallas import tpu_sc as plsc

N_NODES = 50000
N_EDGES = 800000

NCORE = 2
NSUB = 16
NW = NCORE * NSUB
BATCH = 128
ROWS = 200
KSUB = 40
EPAD = NW * ROWS * BATCH
ACC_ROWS = 50176
ZROWS = 112
FLUSH = 3136

_MESH = plsc.VectorSubcoreMesh(core_axis_name="c", subcore_axis_name="s")
_SC_PARAMS = pltpu.CompilerParams(use_tc_tiling_on_sc=False)


def _zero_zbuf(zbuf):
    def body(i, _):
        zbuf[i // 2, pl.ds((i % 2) * 16, 16)] = jnp.zeros((16,), jnp.float32)
        return 0
    lax.fori_loop(0, ZROWS * 2, body, 0, unroll=4)


def _zero_stripe(acc, zbuf, s):
    for z in range(FLUSH // ZROWS):
        pltpu.sync_copy(zbuf, acc.at[pl.ds(s * FLUSH + z * ZROWS, ZROWS)])


def _pipe_kblock(start_g, wait_g, scat):
    start_g(0, 0)

    def tbody(t, _):
        j0 = t * 2
        start_g(j0 + 1, 1)
        wait_g(j0, 0)
        scat(j0, 0)

        @pl.when(t < KSUB // 2 - 1)
        def _():
            start_g(j0 + 2, 0)
        wait_g(j0 + 1, 1)
        scat(j0 + 1, 1)
        return 0
    lax.fori_loop(0, KSUB // 2, tbody, 0)


@functools.partial(
    pl.kernel,
    out_type=jax.ShapeDtypeStruct((4, ACC_ROWS, 32), jnp.float32),
    mesh=_MESH,
    compiler_params=_SC_PARAMS,
    scratch_types=[
        pltpu.VMEM_SHARED((ACC_ROWS, 32), jnp.float32),
        pltpu.VMEM((KSUB, BATCH), jnp.int32),
        pltpu.VMEM((KSUB, BATCH), jnp.int32),
        pltpu.VMEM((2, BATCH, 32), jnp.float32),
        pltpu.VMEM((ZROWS, 32), jnp.float32),
        pltpu.SemaphoreType.DMA,
        pltpu.SemaphoreType.DMA,
    ],
)
def _sc_seg4(src_hbm, dst_hbm, x_hbm, out_hbm,
             acc, srcv, dstv, rows, zbuf, sem_g, sem_s):
    c = lax.axis_index("c")
    s = lax.axis_index("s")
    _zero_zbuf(zbuf)
    for ci in range(2):
        chunk = c * 2 + ci
        _zero_stripe(acc, zbuf, s)
        plsc.subcore_barrier()
        for w in range(2):
            wid = s * 2 + w

            def kbody(k, _):
                pltpu.sync_copy(src_hbm.at[wid].at[pl.ds(k * KSUB, KSUB)], srcv)
                pltpu.sync_copy(dst_hbm.at[wid].at[pl.ds(k * KSUB, KSUB)], dstv)
                _pipe_kblock(
                    lambda j, p: pltpu.async_copy(
                        x_hbm.at[chunk].at[srcv.at[j]], rows.at[p], sem_g),
                    lambda j, p: pltpu.make_async_copy(
                        x_hbm.at[chunk].at[srcv.at[j]], rows.at[p], sem_g).wait(),
                    lambda j, p: pltpu.sync_copy(
                        rows.at[p], acc.at[dstv.at[j]], add=True))
                return 0
            lax.fori_loop(0, ROWS // KSUB, kbody, 0)
        plsc.subcore_barrier()
        pltpu.sync_copy(acc.at[pl.ds(s * FLUSH, FLUSH)],
                        out_hbm.at[chunk].at[pl.ds(s * FLUSH, FLUSH)])
        plsc.subcore_barrier()


@functools.partial(
    pl.kernel,
    out_type=jax.ShapeDtypeStruct((2, ACC_ROWS, 32), jnp.float32),
    mesh=_MESH,
    compiler_params=_SC_PARAMS,
    scratch_types=[
        pltpu.VMEM_SHARED((ACC_ROWS, 32), jnp.float32),
        pltpu.VMEM((KSUB, BATCH), jnp.int32),
        pltpu.VMEM((KSUB, BATCH), jnp.int32),
        pltpu.VMEM((2, BATCH, 32), jnp.float32),
        pltpu.VMEM((ZROWS, 32), jnp.float32),
        pltpu.SemaphoreType.DMA,
        pltpu.SemaphoreType.DMA,
    ],
)
def _sc_seg1(src_hbm, dst_hbm, x_hbm, out_hbm,
             acc, srcv, dstv, rows, zbuf, sem_g, sem_s):
    c = lax.axis_index("c")
    s = lax.axis_index("s")
    _zero_zbuf(zbuf)
    _zero_stripe(acc, zbuf, s)
    plsc.subcore_barrier()
    wid = s * 2 + c

    def kbody(k, _):
        pltpu.sync_copy(src_hbm.at[wid].at[pl.ds(k * KSUB, KSUB)], srcv)
        pltpu.sync_copy(dst_hbm.at[wid].at[pl.ds(k * KSUB, KSUB)], dstv)
        _pipe_kblock(
            lambda j, p: pltpu.async_copy(
                x_hbm.at[srcv.at[j]], rows.at[p], sem_g),
            lambda j, p: pltpu.make_async_copy(
                x_hbm.at[srcv.at[j]], rows.at[p], sem_g).wait(),
            lambda j, p: pltpu.sync_copy(
                rows.at[p], acc.at[dstv.at[j]], add=True))
        return 0
    lax.fori_loop(0, ROWS // KSUB, kbody, 0)
    plsc.subcore_barrier()
    pltpu.sync_copy(acc.at[pl.ds(s * FLUSH, FLUSH)],
                    out_hbm.at[c].at[pl.ds(s * FLUSH, FLUSH)])


@functools.partial(
    pl.kernel,
    out_type=jax.ShapeDtypeStruct((2, ACC_ROWS, 8), jnp.float32),
    mesh=_MESH,
    compiler_params=_SC_PARAMS,
    scratch_types=[
        pltpu.VMEM_SHARED((ACC_ROWS, 8), jnp.float32),
        pltpu.VMEM((KSUB, BATCH), jnp.int32),
        pltpu.VMEM((2, BATCH, 8), jnp.float32),
        pltpu.SemaphoreType.DMA,
    ],
)
def _sc_efagg(dst_hbm, efx_hbm, z8_hbm, out_hbm, acc, dstv, rowbuf, sem):
    c = lax.axis_index("c")
    s = lax.axis_index("s")
    pltpu.sync_copy(z8_hbm, acc.at[pl.ds(s * FLUSH, FLUSH)])
    plsc.subcore_barrier()
    wid = s * 2 + c
    base = wid * (ROWS * BATCH)

    def kbody(k, _):
        pltpu.sync_copy(dst_hbm.at[wid].at[pl.ds(k * KSUB, KSUB)], dstv)
        kb = base + k * KSUB * BATCH
        _pipe_kblock(
            lambda j, p: pltpu.async_copy(
                efx_hbm.at[pl.ds(kb + j * BATCH, BATCH)], rowbuf.at[p], sem),
            lambda j, p: pltpu.make_async_copy(
                efx_hbm.at[pl.ds(kb + j * BATCH, BATCH)], rowbuf.at[p], sem).wait(),
            lambda j, p: pltpu.sync_copy(
                rowbuf.at[p], acc.at[dstv.at[j]], add=True))
        return 0
    lax.fori_loop(0, ROWS // KSUB, kbody, 0)
    plsc.subcore_barrier()
    pltpu.sync_copy(acc.at[pl.ds(s * FLUSH, FLUSH)],
                    out_hbm.at[c].at[pl.ds(s * FLUSH, FLUSH)])


BN = 1000
NBLK = 50


def _tc_layer_body(ns, with_res, *refs):
    if with_res:
        s_ref, ws_ref, ef_ref, e8_ref, gb_ref, r_ref, out_ref = refs
    else:
        s_ref, ws_ref, ef_ref, e8_ref, gb_ref, out_ref = refs
        r_ref = None
    agg = jnp.dot(s_ref[0], ws_ref[0], preferred_element_type=jnp.float32)
    for n in range(1, ns):
        agg += jnp.dot(s_ref[n], ws_ref[n], preferred_element_type=jnp.float32)
    agg += jnp.dot(ef_ref[0] + ef_ref[1], e8_ref[...],
                   preferred_element_type=jnp.float32)
    h = jnp.maximum(agg, 0.0)
    mu = jnp.mean(h, axis=1, keepdims=True)
    var = jnp.mean((h - mu) ** 2, axis=1, keepdims=True)
    h = (h - mu) / jnp.sqrt(var + 1e-5) * gb_ref[0:1] + gb_ref[1:2]
    for ch in range(4):
        piece = h[:, ch * 32:(ch + 1) * 32]
        if r_ref is not None:
            piece = piece + r_ref[ch]
        out_ref[ch] = piece


def _make_tc_layer(ns, with_res):
    in_specs = [
        pl.BlockSpec((ns, BN, 32), lambda i: (0, i, 0)),
        pl.BlockSpec((ns, 32, 128), lambda i: (0, 0, 0)),
        pl.BlockSpec((2, BN, 8), lambda i: (0, i, 0)),
        pl.BlockSpec((8, 128), lambda i: (0, 0)),
        pl.BlockSpec((2, 128), lambda i: (0, 0)),
    ]
    if with_res:
        in_specs.append(pl.BlockSpec((4, BN, 32), lambda i: (0, i, 0)))
    return pl.pallas_call(
        functools.partial(_tc_layer_body, ns, with_res),
        grid=(NBLK,),
        in_specs=in_specs,
        out_specs=pl.BlockSpec((4, BN, 32), lambda i: (0, i, 0)),
        out_shape=jax.ShapeDtypeStruct((4, N_NODES, 32), jnp.float32),
    )


_tc_layer0 = _make_tc_layer(2, False)
_tc_layer1 = _make_tc_layer(4, True)
_tc_layer2 = _make_tc_layer(4, False)


def _ln_row(h, g, b):
    mu = jnp.mean(h, axis=1, keepdims=True)
    var = jnp.mean((h - mu) ** 2, axis=1, keepdims=True)
    return (h - mu) / jnp.sqrt(var + 1e-5) * g + b


def _attn_head_body(x_ref, pe_ref, wq_ref, bq_ref, wk_ref, bk_ref,
                    wv_ref, bv_ref, wo_ref, bo_ref, g_ref, hm_ref,
                    w1a_ref, w1b_ref, b1_ref, g1_ref, n1_ref,
                    w2_ref, b2_ref, g2_ref, n2_ref,
                    w3_ref, b3_ref, g3_ref, n3_ref,
                    w4_ref, b4_ref, out_ref,
                    qs, m_sc, l_sc, acc_sc):
    i = pl.program_id(0)

    @pl.when(i == 0)
    def _init():
        q = jnp.dot(pe_ref[...], wq_ref[...],
                    preferred_element_type=jnp.float32) + bq_ref[...]
        qs[...] = q / jnp.sqrt(32.0)
        m_sc[...] = jnp.full((4, 1), -1e30, jnp.float32)
        l_sc[...] = jnp.zeros((4, 1), jnp.float32)
        acc_sc[...] = jnp.zeros((4, 128), jnp.float32)

    k = jnp.dot(x_ref[0], wk_ref[0:32], preferred_element_type=jnp.float32)
    v = jnp.dot(x_ref[0], wv_ref[0:32], preferred_element_type=jnp.float32)
    for ch in range(1, 4):
        k += jnp.dot(x_ref[ch], wk_ref[ch * 32:(ch + 1) * 32],
                     preferred_element_type=jnp.float32)
        v += jnp.dot(x_ref[ch], wv_ref[ch * 32:(ch + 1) * 32],
                     preferred_element_type=jnp.float32)
    k = k + bk_ref[...]
    v = v + bv_ref[...]
    kq = k * qs[...]
    sT = lax.dot_general(g_ref[...], kq, (((0,), (1,)), ((), ())),
                         preferred_element_type=jnp.float32)
    m_blk = jnp.max(sT, axis=1, keepdims=True)
    m_new = jnp.maximum(m_sc[...], m_blk)
    alpha = jnp.exp(m_sc[...] - m_new)
    p = jnp.exp(sT - m_new)
    l_sc[...] = l_sc[...] * alpha + jnp.sum(p, axis=1, keepdims=True)
    acc_sc[...] = acc_sc[...] * alpha + lax.dot_general(
        p, v, (((1,), (0,)), ((), ())), preferred_element_type=jnp.float32)
    m_sc[...] = m_new

    @pl.when(i == NBLK - 1)
    def _final():
        att = acc_sc[...] / l_sc[...]
        att1 = jnp.sum(att * hm_ref[...], axis=0, keepdims=True)
        o = jnp.dot(att1, wo_ref[...],
                    preferred_element_type=jnp.float32) + bo_ref[...]
        h = (jnp.dot(pe_ref[...], w1a_ref[...], preferred_element_type=jnp.float32)
             + jnp.dot(o, w1b_ref[...], preferred_element_type=jnp.float32)
             + b1_ref[...])
        h = jnp.maximum(_ln_row(h, g1_ref[...], n1_ref[...]), 0.0)
        h = jnp.dot(h, w2_ref[...], preferred_element_type=jnp.float32) + b2_ref[...]
        h = jnp.maximum(_ln_row(h, g2_ref[...], n2_ref[...]), 0.0)
        h = jnp.dot(h, w3_ref[...], preferred_element_type=jnp.float32) + b3_ref[...]
        h = jnp.maximum(_ln_row(h, g3_ref[...], n3_ref[...]), 0.0)
        pred = jnp.dot(h, w4_ref[...], preferred_element_type=jnp.float32) + b4_ref[...]
        out_ref[...] = pred


def _full(shape):
    nd = len(shape)
    return pl.BlockSpec(shape, lambda i: (0,) * nd)


_attn_head = pl.pallas_call(
    _attn_head_body,
    grid=(NBLK,),
    in_specs=[
        pl.BlockSpec((4, BN, 32), lambda i: (0, i, 0)),
        _full((1, 480)), _full((480, 128)), _full((1, 128)),
        _full((128, 128)), _full((1, 128)),
        _full((128, 128)), _full((1, 128)),
        _full((128, 128)), _full((1, 128)),
        _full((128, 4)), _full((4, 128)),
        _full((480, 512)), _full((128, 512)), _full((1, 512)),
        _full((1, 512)), _full((1, 512)),
        _full((512, 256)), _full((1, 256)), _full((1, 256)), _full((1, 256)),
        _full((256, 128)), _full((1, 128)), _full((1, 128)), _full((1, 128)),
        _full((128, 1)), _full((1, 1)),
    ],
    out_specs=_full((1, 1)),
    out_shape=jax.ShapeDtypeStruct((1, 1), jnp.float32),
    scratch_shapes=[
        pltpu.VMEM((1, 128), jnp.float32),
        pltpu.VMEM((4, 1), jnp.float32),
        pltpu.VMEM((4, 1), jnp.float32),
        pltpu.VMEM((4, 128), jnp.float32),
    ],
)


def kernel(proteinEmbedding, nodeFeatures, edgeIndex, edgeFeatures, params):
    f32 = jnp.float32
    src = edgeIndex[:, 0]
    dst = edgeIndex[:, 1]
    pad = EPAD - N_EDGES
    srcp = jnp.concatenate([src, jnp.zeros((pad,), jnp.int32)]).reshape(NW, ROWS, BATCH)
    dstp = jnp.concatenate(
        [dst, jnp.full((pad,), N_NODES, jnp.int32)]).reshape(NW, ROWS, BATCH)
    efx = jnp.concatenate(
        [edgeFeatures, jnp.zeros((N_EDGES, 1), f32), jnp.ones((N_EDGES, 1), f32)], 1)
    efxp = jnp.concatenate([efx, jnp.zeros((pad, 8), f32)], 0)
    z8 = jnp.zeros((FLUSH, 8), f32)
    x0 = jnp.pad(nodeFeatures, ((0, 0), (0, 13)))

    gcn = params['gcn']
    ws = []
    e8s = []
    gbs = []
    for i, p in enumerate(gcn):
        if i == 0:
            w = jnp.pad(p['W'], ((0, 13), (0, 0)))
            ws.append(jnp.stack([w, w]))
        else:
            ws.append(p['W'].reshape(4, 32, 128))
        e8s.append(jnp.concatenate(
            [p['We'], jnp.zeros((1, 128), f32), (p['b'] + p['be'])[None]], 0))
        gbs.append(jnp.stack([p['g'], p['bn']]))

    EF = _sc_efagg(dstp, efxp, z8)
    S0 = _sc_seg1(srcp, dstp, x0)
    h0 = _tc_layer0(S0, ws[0], EF, e8s[0], gbs[0])
    S1 = _sc_seg4(srcp, dstp, h0)
    h1 = _tc_layer1(S1, ws[1], EF, e8s[1], gbs[1], h0)
    S2 = _sc_seg4(srcp, dstp, h1)
    h2 = _tc_layer2(S2, ws[2], EF, e8s[2], gbs[2])

    mlp = params['mlp']
    G = (jnp.arange(128)[:, None] // 32 == jnp.arange(4)[None, :]).astype(f32)
    HM = G.T
    pred = _attn_head(
        h2, proteinEmbedding[None], params['Wq'], params['bq'][None],
        params['Wk'], params['bk'][None], params['Wv'], params['bv'][None],
        params['Wo'], params['bo'][None], G, HM,
        mlp[0]['W'][:480], mlp[0]['W'][480:], mlp[0]['b'][None],
        mlp[0]['g'][None], mlp[0]['bn'][None],
        mlp[1]['W'], mlp[1]['b'][None], mlp[1]['g'][None], mlp[1]['bn'][None],
        mlp[2]['W'], mlp[2]['b'][None], mlp[2]['g'][None], mlp[2]['bn'][None],
        mlp[3]['W'], mlp[3]['b'][None],
    )
    return pred.reshape(1)


# speedup vs baseline: 3.6267x; 1.0001x over previous
import functools

import jax
import jax.numpy as jnp
from jax import lax
from jax.experimental import pallas as pl
from jax.experimental.pallas import tpu as pltpu
from jax.experimental.pallas import tpu_sc as plsc

N_NODES = 50000
N_EDGES = 800000

NCORE = 2
NSUB = 16
NW = NCORE * NSUB
BATCH = 128
ROWS = 200
KSUB = 40
EPAD = NW * ROWS * BATCH
ACC_ROWS = 50176
ZROWS = 112
FLUSH = 3136

_MESH = plsc.VectorSubcoreMesh(core_axis_name="c", subcore_axis_name="s")
_SC_PARAMS = pltpu.CompilerParams(use_tc_tiling_on_sc=False)


def _zero_zbuf(zbuf):
    def body(i, _):
        zbuf[i // 2, pl.ds((i % 2) * 16, 16)] = jnp.zeros((16,), jnp.float32)
        return 0
    lax.fori_loop(0, ZROWS * 2, body, 0, unroll=4)


def _zero_stripe(acc, zbuf, s):
    for z in range(FLUSH // ZROWS):
        pltpu.sync_copy(zbuf, acc.at[pl.ds(s * FLUSH + z * ZROWS, ZROWS)])


def _pipe_kblock(start_g, wait_g, scat):
    start_g(0, 0)

    def tbody(t, _):
        j0 = t * 2
        start_g(j0 + 1, 1)
        wait_g(j0, 0)
        scat(j0, 0)

        @pl.when(t < KSUB // 2 - 1)
        def _():
            start_g(j0 + 2, 0)
        wait_g(j0 + 1, 1)
        scat(j0 + 1, 1)
        return 0
    lax.fori_loop(0, KSUB // 2, tbody, 0)


@functools.partial(
    pl.kernel,
    out_type=jax.ShapeDtypeStruct((4, ACC_ROWS, 32), jnp.float32),
    mesh=_MESH,
    compiler_params=_SC_PARAMS,
    scratch_types=[
        pltpu.VMEM_SHARED((ACC_ROWS, 32), jnp.float32),
        pltpu.VMEM((KSUB, BATCH), jnp.int32),
        pltpu.VMEM((KSUB, BATCH), jnp.int32),
        pltpu.VMEM((2, BATCH, 32), jnp.float32),
        pltpu.VMEM((ZROWS, 32), jnp.float32),
        pltpu.SemaphoreType.DMA,
        pltpu.SemaphoreType.DMA,
    ],
)
def _sc_seg4(src_hbm, dst_hbm, x_hbm, out_hbm,
             acc, srcv, dstv, rows, zbuf, sem_g, sem_s):
    c = lax.axis_index("c")
    s = lax.axis_index("s")
    _zero_zbuf(zbuf)
    for ci in range(2):
        chunk = c * 2 + ci
        _zero_stripe(acc, zbuf, s)
        plsc.subcore_barrier()
        for w in range(2):
            wid = s * 2 + w

            def kbody(k, _):
                pltpu.sync_copy(src_hbm.at[wid].at[pl.ds(k * KSUB, KSUB)], srcv)
                pltpu.sync_copy(dst_hbm.at[wid].at[pl.ds(k * KSUB, KSUB)], dstv)
                sems = (sem_g, sem_s)
                _pipe_kblock(
                    lambda j, p: pltpu.async_copy(
                        x_hbm.at[chunk].at[srcv.at[j]], rows.at[p], sems[p]),
                    lambda j, p: pltpu.make_async_copy(
                        x_hbm.at[chunk].at[srcv.at[j]], rows.at[p], sems[p]).wait(),
                    lambda j, p: pltpu.sync_copy(
                        rows.at[p], acc.at[dstv.at[j]], add=True))
                return 0
            lax.fori_loop(0, ROWS // KSUB, kbody, 0)
        plsc.subcore_barrier()
        pltpu.sync_copy(acc.at[pl.ds(s * FLUSH, FLUSH)],
                        out_hbm.at[chunk].at[pl.ds(s * FLUSH, FLUSH)])
        plsc.subcore_barrier()


@functools.partial(
    pl.kernel,
    out_type=jax.ShapeDtypeStruct((2, ACC_ROWS, 32), jnp.float32),
    mesh=_MESH,
    compiler_params=_SC_PARAMS,
    scratch_types=[
        pltpu.VMEM_SHARED((ACC_ROWS, 32), jnp.float32),
        pltpu.VMEM((KSUB, BATCH), jnp.int32),
        pltpu.VMEM((KSUB, BATCH), jnp.int32),
        pltpu.VMEM((2, BATCH, 32), jnp.float32),
        pltpu.VMEM((ZROWS, 32), jnp.float32),
        pltpu.SemaphoreType.DMA,
        pltpu.SemaphoreType.DMA,
    ],
)
def _sc_seg1(src_hbm, dst_hbm, x_hbm, out_hbm,
             acc, srcv, dstv, rows, zbuf, sem_g, sem_s):
    c = lax.axis_index("c")
    s = lax.axis_index("s")
    _zero_zbuf(zbuf)
    _zero_stripe(acc, zbuf, s)
    plsc.subcore_barrier()
    wid = s * 2 + c

    def kbody(k, _):
        pltpu.sync_copy(src_hbm.at[wid].at[pl.ds(k * KSUB, KSUB)], srcv)
        pltpu.sync_copy(dst_hbm.at[wid].at[pl.ds(k * KSUB, KSUB)], dstv)
        sems = (sem_g, sem_s)
        _pipe_kblock(
            lambda j, p: pltpu.async_copy(
                x_hbm.at[srcv.at[j]], rows.at[p], sems[p]),
            lambda j, p: pltpu.make_async_copy(
                x_hbm.at[srcv.at[j]], rows.at[p], sems[p]).wait(),
            lambda j, p: pltpu.sync_copy(
                rows.at[p], acc.at[dstv.at[j]], add=True))
        return 0
    lax.fori_loop(0, ROWS // KSUB, kbody, 0)
    plsc.subcore_barrier()
    pltpu.sync_copy(acc.at[pl.ds(s * FLUSH, FLUSH)],
                    out_hbm.at[c].at[pl.ds(s * FLUSH, FLUSH)])


@functools.partial(
    pl.kernel,
    out_type=jax.ShapeDtypeStruct((2, ACC_ROWS, 8), jnp.float32),
    mesh=_MESH,
    compiler_params=_SC_PARAMS,
    scratch_types=[
        pltpu.VMEM_SHARED((ACC_ROWS, 8), jnp.float32),
        pltpu.VMEM((KSUB, BATCH), jnp.int32),
        pltpu.VMEM((2, BATCH, 8), jnp.float32),
        pltpu.SemaphoreType.DMA,
        pltpu.SemaphoreType.DMA,
    ],
)
def _sc_efagg(dst_hbm, efx_hbm, z8_hbm, out_hbm, acc, dstv, rowbuf, sem, sem2):
    c = lax.axis_index("c")
    s = lax.axis_index("s")
    pltpu.sync_copy(z8_hbm, acc.at[pl.ds(s * FLUSH, FLUSH)])
    plsc.subcore_barrier()
    wid = s * 2 + c
    base = wid * (ROWS * BATCH)

    def kbody(k, _):
        pltpu.sync_copy(dst_hbm.at[wid].at[pl.ds(k * KSUB, KSUB)], dstv)
        kb = base + k * KSUB * BATCH
        sems = (sem, sem2)
        _pipe_kblock(
            lambda j, p: pltpu.async_copy(
                efx_hbm.at[pl.ds(kb + j * BATCH, BATCH)], rowbuf.at[p], sems[p]),
            lambda j, p: pltpu.make_async_copy(
                efx_hbm.at[pl.ds(kb + j * BATCH, BATCH)], rowbuf.at[p], sems[p]).wait(),
            lambda j, p: pltpu.sync_copy(
                rowbuf.at[p], acc.at[dstv.at[j]], add=True))
        return 0
    lax.fori_loop(0, ROWS // KSUB, kbody, 0)
    plsc.subcore_barrier()
    pltpu.sync_copy(acc.at[pl.ds(s * FLUSH, FLUSH)],
                    out_hbm.at[c].at[pl.ds(s * FLUSH, FLUSH)])


BN = 1000
NBLK = 50


def _tc_layer_body(ns, with_res, *refs):
    if with_res:
        s_ref, ws_ref, ef_ref, e8_ref, gb_ref, r_ref, out_ref = refs
    else:
        s_ref, ws_ref, ef_ref, e8_ref, gb_ref, out_ref = refs
        r_ref = None
    agg = jnp.dot(s_ref[0], ws_ref[0], preferred_element_type=jnp.float32)
    for n in range(1, ns):
        agg += jnp.dot(s_ref[n], ws_ref[n], preferred_element_type=jnp.float32)
    agg += jnp.dot(ef_ref[0] + ef_ref[1], e8_ref[...],
                   preferred_element_type=jnp.float32)
    h = jnp.maximum(agg, 0.0)
    mu = jnp.mean(h, axis=1, keepdims=True)
    var = jnp.mean((h - mu) ** 2, axis=1, keepdims=True)
    h = (h - mu) / jnp.sqrt(var + 1e-5) * gb_ref[0:1] + gb_ref[1:2]
    for ch in range(4):
        piece = h[:, ch * 32:(ch + 1) * 32]
        if r_ref is not None:
            piece = piece + r_ref[ch]
        out_ref[ch] = piece


def _make_tc_layer(ns, with_res):
    in_specs = [
        pl.BlockSpec((ns, BN, 32), lambda i: (0, i, 0)),
        pl.BlockSpec((ns, 32, 128), lambda i: (0, 0, 0)),
        pl.BlockSpec((2, BN, 8), lambda i: (0, i, 0)),
        pl.BlockSpec((8, 128), lambda i: (0, 0)),
        pl.BlockSpec((2, 128), lambda i: (0, 0)),
    ]
    if with_res:
        in_specs.append(pl.BlockSpec((4, BN, 32), lambda i: (0, i, 0)))
    return pl.pallas_call(
        functools.partial(_tc_layer_body, ns, with_res),
        grid=(NBLK,),
        in_specs=in_specs,
        out_specs=pl.BlockSpec((4, BN, 32), lambda i: (0, i, 0)),
        out_shape=jax.ShapeDtypeStruct((4, N_NODES, 32), jnp.float32),
    )


_tc_layer0 = _make_tc_layer(2, False)
_tc_layer1 = _make_tc_layer(4, True)
_tc_layer2 = _make_tc_layer(4, False)


def _ln_row(h, g, b):
    mu = jnp.mean(h, axis=1, keepdims=True)
    var = jnp.mean((h - mu) ** 2, axis=1, keepdims=True)
    return (h - mu) / jnp.sqrt(var + 1e-5) * g + b


def _attn_head_body(x_ref, pe_ref, wq_ref, bq_ref, wk_ref, bk_ref,
                    wv_ref, bv_ref, wo_ref, bo_ref, g_ref, hm_ref,
                    w1a_ref, w1b_ref, b1_ref, g1_ref, n1_ref,
                    w2_ref, b2_ref, g2_ref, n2_ref,
                    w3_ref, b3_ref, g3_ref, n3_ref,
                    w4_ref, b4_ref, out_ref,
                    qs, m_sc, l_sc, acc_sc):
    i = pl.program_id(0)

    @pl.when(i == 0)
    def _init():
        q = jnp.dot(pe_ref[...], wq_ref[...],
                    preferred_element_type=jnp.float32) + bq_ref[...]
        qs[...] = q / jnp.sqrt(32.0)
        m_sc[...] = jnp.full((4, 1), -1e30, jnp.float32)
        l_sc[...] = jnp.zeros((4, 1), jnp.float32)
        acc_sc[...] = jnp.zeros((4, 128), jnp.float32)

    k = jnp.dot(x_ref[0], wk_ref[0:32], preferred_element_type=jnp.float32)
    v = jnp.dot(x_ref[0], wv_ref[0:32], preferred_element_type=jnp.float32)
    for ch in range(1, 4):
        k += jnp.dot(x_ref[ch], wk_ref[ch * 32:(ch + 1) * 32],
                     preferred_element_type=jnp.float32)
        v += jnp.dot(x_ref[ch], wv_ref[ch * 32:(ch + 1) * 32],
                     preferred_element_type=jnp.float32)
    k = k + bk_ref[...]
    v = v + bv_ref[...]
    kq = k * qs[...]
    sT = lax.dot_general(g_ref[...], kq, (((0,), (1,)), ((), ())),
                         preferred_element_type=jnp.float32)
    m_blk = jnp.max(sT, axis=1, keepdims=True)
    m_new = jnp.maximum(m_sc[...], m_blk)
    alpha = jnp.exp(m_sc[...] - m_new)
    p = jnp.exp(sT - m_new)
    l_sc[...] = l_sc[...] * alpha + jnp.sum(p, axis=1, keepdims=True)
    acc_sc[...] = acc_sc[...] * alpha + lax.dot_general(
        p, v, (((1,), (0,)), ((), ())), preferred_element_type=jnp.float32)
    m_sc[...] = m_new

    @pl.when(i == NBLK - 1)
    def _final():
        att = acc_sc[...] / l_sc[...]
        att1 = jnp.sum(att * hm_ref[...], axis=0, keepdims=True)
        o = jnp.dot(att1, wo_ref[...],
                    preferred_element_type=jnp.float32) + bo_ref[...]
        h = (jnp.dot(pe_ref[...], w1a_ref[...], preferred_element_type=jnp.float32)
             + jnp.dot(o, w1b_ref[...], preferred_element_type=jnp.float32)
             + b1_ref[...])
        h = jnp.maximum(_ln_row(h, g1_ref[...], n1_ref[...]), 0.0)
        h = jnp.dot(h, w2_ref[...], preferred_element_type=jnp.float32) + b2_ref[...]
        h = jnp.maximum(_ln_row(h, g2_ref[...], n2_ref[...]), 0.0)
        h = jnp.dot(h, w3_ref[...], preferred_element_type=jnp.float32) + b3_ref[...]
        h = jnp.maximum(_ln_row(h, g3_ref[...], n3_ref[...]), 0.0)
        pred = jnp.dot(h, w4_ref[...], preferred_element_type=jnp.float32) + b4_ref[...]
        out_ref[...] = pred


def _full(shape):
    nd = len(shape)
    return pl.BlockSpec(shape, lambda i: (0,) * nd)


_attn_head = pl.pallas_call(
    _attn_head_body,
    grid=(NBLK,),
    in_specs=[
        pl.BlockSpec((4, BN, 32), lambda i: (0, i, 0)),
        _full((1, 480)), _full((480, 128)), _full((1, 128)),
        _full((128, 128)), _full((1, 128)),
        _full((128, 128)), _full((1, 128)),
        _full((128, 128)), _full((1, 128)),
        _full((128, 4)), _full((4, 128)),
        _full((480, 512)), _full((128, 512)), _full((1, 512)),
        _full((1, 512)), _full((1, 512)),
        _full((512, 256)), _full((1, 256)), _full((1, 256)), _full((1, 256)),
        _full((256, 128)), _full((1, 128)), _full((1, 128)), _full((1, 128)),
        _full((128, 1)), _full((1, 1)),
    ],
    out_specs=_full((1, 1)),
    out_shape=jax.ShapeDtypeStruct((1, 1), jnp.float32),
    scratch_shapes=[
        pltpu.VMEM((1, 128), jnp.float32),
        pltpu.VMEM((4, 1), jnp.float32),
        pltpu.VMEM((4, 1), jnp.float32),
        pltpu.VMEM((4, 128), jnp.float32),
    ],
)


def kernel(proteinEmbedding, nodeFeatures, edgeIndex, edgeFeatures, params):
    f32 = jnp.float32
    src = edgeIndex[:, 0]
    dst = edgeIndex[:, 1]
    pad = EPAD - N_EDGES
    srcp = jnp.concatenate([src, jnp.zeros((pad,), jnp.int32)]).reshape(NW, ROWS, BATCH)
    dstp = jnp.concatenate(
        [dst, jnp.full((pad,), N_NODES, jnp.int32)]).reshape(NW, ROWS, BATCH)
    efx = jnp.concatenate(
        [edgeFeatures, jnp.zeros((N_EDGES, 1), f32), jnp.ones((N_EDGES, 1), f32)], 1)
    efxp = jnp.concatenate([efx, jnp.zeros((pad, 8), f32)], 0)
    z8 = jnp.zeros((FLUSH, 8), f32)
    x0 = jnp.pad(nodeFeatures, ((0, 0), (0, 13)))

    gcn = params['gcn']
    ws = []
    e8s = []
    gbs = []
    for i, p in enumerate(gcn):
        if i == 0:
            w = jnp.pad(p['W'], ((0, 13), (0, 0)))
            ws.append(jnp.stack([w, w]))
        else:
            ws.append(p['W'].reshape(4, 32, 128))
        e8s.append(jnp.concatenate(
            [p['We'], jnp.zeros((1, 128), f32), (p['b'] + p['be'])[None]], 0))
        gbs.append(jnp.stack([p['g'], p['bn']]))

    EF = _sc_efagg(dstp, efxp, z8)
    S0 = _sc_seg1(srcp, dstp, x0)
    h0 = _tc_layer0(S0, ws[0], EF, e8s[0], gbs[0])
    S1 = _sc_seg4(srcp, dstp, h0)
    h1 = _tc_layer1(S1, ws[1], EF, e8s[1], gbs[1], h0)
    S2 = _sc_seg4(srcp, dstp, h1)
    h2 = _tc_layer2(S2, ws[2], EF, e8s[2], gbs[2])

    mlp = params['mlp']
    G = (jnp.arange(128)[:, None] // 32 == jnp.arange(4)[None, :]).astype(f32)
    HM = G.T
    pred = _attn_head(
        h2, proteinEmbedding[None], params['Wq'], params['bq'][None],
        params['Wk'], params['bk'][None], params['Wv'], params['bv'][None],
        params['Wo'], params['bo'][None], G, HM,
        mlp[0]['W'][:480], mlp[0]['W'][480:], mlp[0]['b'][None],
        mlp[0]['g'][None], mlp[0]['bn'][None],
        mlp[1]['W'], mlp[1]['b'][None], mlp[1]['g'][None], mlp[1]['bn'][None],
        mlp[2]['W'], mlp[2]['b'][None], mlp[2]['g'][None], mlp[2]['bn'][None],
        mlp[3]['W'], mlp[3]['b'][None],
    )
    return pred.reshape(1)


# speedup vs baseline: 3.8440x; 1.0599x over previous
import functools

import jax
import jax.numpy as jnp
from jax import lax
from jax.experimental import pallas as pl
from jax.experimental.pallas import tpu as pltpu
from jax.experimental.pallas import tpu_sc as plsc

N_NODES = 50000
N_EDGES = 800000

NCORE = 2
NSUB = 16
NW = NCORE * NSUB
BATCH = 128
ROWS = 200
KSUB = 40
EPAD = NW * ROWS * BATCH
ACC_ROWS = 50176
ZROWS = 56
FLUSH = 3136

_MESH = plsc.VectorSubcoreMesh(core_axis_name="c", subcore_axis_name="s")
_SC_PARAMS = pltpu.CompilerParams(use_tc_tiling_on_sc=False)


def _zero_zbuf(zbuf):
    def body(i, _):
        zbuf[i // 2, pl.ds((i % 2) * 16, 16)] = jnp.zeros((16,), jnp.float32)
        return 0
    lax.fori_loop(0, ZROWS * 2, body, 0, unroll=4)


def _zero_stripe(acc, zbuf, s):
    for z in range(FLUSH // ZROWS):
        pltpu.sync_copy(zbuf, acc.at[pl.ds(s * FLUSH + z * ZROWS, ZROWS)])


def _pipe_kblock(start_g, wait_g, start_s, wait_s):
    start_g(0, 0)
    start_g(1, 1)

    def tbody(t, _):
        for q in range(4):
            j = t * 4 + q
            if q < 2:
                @pl.when(t > 0)
                def _(j=j, q=q):
                    wait_s(j - 2, (q + 2) % 4)
                start_g(j + 2, (q + 2) % 4)
            else:
                wait_s(j - 2, (q + 2) % 4)

                @pl.when(t < KSUB // 4 - 1)
                def _(j=j, q=q):
                    start_g(j + 2, (q + 2) % 4)
            wait_g(j, q)
            start_s(j, q)
        return 0
    lax.fori_loop(0, KSUB // 4, tbody, 0)
    wait_s(KSUB - 2, 2)
    wait_s(KSUB - 1, 3)


@functools.partial(
    pl.kernel,
    out_type=jax.ShapeDtypeStruct((4, ACC_ROWS, 32), jnp.float32),
    mesh=_MESH,
    compiler_params=_SC_PARAMS,
    scratch_types=[
        pltpu.VMEM_SHARED((ACC_ROWS, 32), jnp.float32),
        pltpu.VMEM((KSUB, BATCH), jnp.int32),
        pltpu.VMEM((KSUB, BATCH), jnp.int32),
        pltpu.VMEM((4, BATCH, 32), jnp.float32),
        pltpu.VMEM((ZROWS, 32), jnp.float32),
        [pltpu.SemaphoreType.DMA] * 4,
        [pltpu.SemaphoreType.DMA] * 4,
    ],
)
def _sc_seg4(src_hbm, dst_hbm, x_hbm, out_hbm,
             acc, srcv, dstv, rows, zbuf, gsem, ssem):
    c = lax.axis_index("c")
    s = lax.axis_index("s")
    _zero_zbuf(zbuf)
    for ci in range(2):
        chunk = c * 2 + ci
        _zero_stripe(acc, zbuf, s)
        plsc.subcore_barrier()
        for w in range(2):
            wid = s * 2 + w

            def kbody(k, _):
                pltpu.sync_copy(src_hbm.at[wid].at[pl.ds(k * KSUB, KSUB)], srcv)
                pltpu.sync_copy(dst_hbm.at[wid].at[pl.ds(k * KSUB, KSUB)], dstv)
                _pipe_kblock(
                    lambda j, p: pltpu.async_copy(
                        x_hbm.at[chunk].at[srcv.at[j]], rows.at[p], gsem[p]),
                    lambda j, p: pltpu.make_async_copy(
                        x_hbm.at[chunk].at[srcv.at[j]], rows.at[p], gsem[p]).wait(),
                    lambda j, p: pltpu.async_copy(
                        rows.at[p], acc.at[dstv.at[j]], ssem[p], add=True),
                    lambda j, p: pltpu.make_async_copy(
                        rows.at[p], acc.at[dstv.at[j]], ssem[p]).wait())
                return 0
            lax.fori_loop(0, ROWS // KSUB, kbody, 0)
        plsc.subcore_barrier()
        pltpu.sync_copy(acc.at[pl.ds(s * FLUSH, FLUSH)],
                        out_hbm.at[chunk].at[pl.ds(s * FLUSH, FLUSH)])
        plsc.subcore_barrier()


@functools.partial(
    pl.kernel,
    out_type=jax.ShapeDtypeStruct((2, ACC_ROWS, 32), jnp.float32),
    mesh=_MESH,
    compiler_params=_SC_PARAMS,
    scratch_types=[
        pltpu.VMEM_SHARED((ACC_ROWS, 32), jnp.float32),
        pltpu.VMEM((KSUB, BATCH), jnp.int32),
        pltpu.VMEM((KSUB, BATCH), jnp.int32),
        pltpu.VMEM((4, BATCH, 32), jnp.float32),
        pltpu.VMEM((ZROWS, 32), jnp.float32),
        [pltpu.SemaphoreType.DMA] * 4,
        [pltpu.SemaphoreType.DMA] * 4,
    ],
)
def _sc_seg1(src_hbm, dst_hbm, x_hbm, out_hbm,
             acc, srcv, dstv, rows, zbuf, gsem, ssem):
    c = lax.axis_index("c")
    s = lax.axis_index("s")
    _zero_zbuf(zbuf)
    _zero_stripe(acc, zbuf, s)
    plsc.subcore_barrier()
    wid = s * 2 + c

    def kbody(k, _):
        pltpu.sync_copy(src_hbm.at[wid].at[pl.ds(k * KSUB, KSUB)], srcv)
        pltpu.sync_copy(dst_hbm.at[wid].at[pl.ds(k * KSUB, KSUB)], dstv)
        _pipe_kblock(
            lambda j, p: pltpu.async_copy(
                x_hbm.at[srcv.at[j]], rows.at[p], gsem[p]),
            lambda j, p: pltpu.make_async_copy(
                x_hbm.at[srcv.at[j]], rows.at[p], gsem[p]).wait(),
            lambda j, p: pltpu.async_copy(
                rows.at[p], acc.at[dstv.at[j]], ssem[p], add=True),
            lambda j, p: pltpu.make_async_copy(
                rows.at[p], acc.at[dstv.at[j]], ssem[p]).wait())
        return 0
    lax.fori_loop(0, ROWS // KSUB, kbody, 0)
    plsc.subcore_barrier()
    pltpu.sync_copy(acc.at[pl.ds(s * FLUSH, FLUSH)],
                    out_hbm.at[c].at[pl.ds(s * FLUSH, FLUSH)])


@functools.partial(
    pl.kernel,
    out_type=jax.ShapeDtypeStruct((2, ACC_ROWS, 8), jnp.float32),
    mesh=_MESH,
    compiler_params=_SC_PARAMS,
    scratch_types=[
        pltpu.VMEM_SHARED((ACC_ROWS, 8), jnp.float32),
        pltpu.VMEM((KSUB, BATCH), jnp.int32),
        pltpu.VMEM((4, BATCH, 8), jnp.float32),
        [pltpu.SemaphoreType.DMA] * 4,
        [pltpu.SemaphoreType.DMA] * 4,
    ],
)
def _sc_efagg(dst_hbm, efx_hbm, z8_hbm, out_hbm, acc, dstv, rowbuf, gsem, ssem):
    c = lax.axis_index("c")
    s = lax.axis_index("s")
    pltpu.sync_copy(z8_hbm, acc.at[pl.ds(s * FLUSH, FLUSH)])
    plsc.subcore_barrier()
    wid = s * 2 + c
    base = wid * (ROWS * BATCH)

    def kbody(k, _):
        pltpu.sync_copy(dst_hbm.at[wid].at[pl.ds(k * KSUB, KSUB)], dstv)
        kb = base + k * KSUB * BATCH
        _pipe_kblock(
            lambda j, p: pltpu.async_copy(
                efx_hbm.at[pl.ds(kb + j * BATCH, BATCH)], rowbuf.at[p], gsem[p]),
            lambda j, p: pltpu.make_async_copy(
                efx_hbm.at[pl.ds(kb + j * BATCH, BATCH)], rowbuf.at[p], gsem[p]).wait(),
            lambda j, p: pltpu.async_copy(
                rowbuf.at[p], acc.at[dstv.at[j]], ssem[p], add=True),
            lambda j, p: pltpu.make_async_copy(
                rowbuf.at[p], acc.at[dstv.at[j]], ssem[p]).wait())
        return 0
    lax.fori_loop(0, ROWS // KSUB, kbody, 0)
    plsc.subcore_barrier()
    pltpu.sync_copy(acc.at[pl.ds(s * FLUSH, FLUSH)],
                    out_hbm.at[c].at[pl.ds(s * FLUSH, FLUSH)])


BN = 1000
NBLK = 50


def _tc_layer_body(ns, with_res, *refs):
    if with_res:
        s_ref, ws_ref, ef_ref, e8_ref, gb_ref, r_ref, out_ref = refs
    else:
        s_ref, ws_ref, ef_ref, e8_ref, gb_ref, out_ref = refs
        r_ref = None
    agg = jnp.dot(s_ref[0], ws_ref[0], preferred_element_type=jnp.float32)
    for n in range(1, ns):
        agg += jnp.dot(s_ref[n], ws_ref[n], preferred_element_type=jnp.float32)
    agg += jnp.dot(ef_ref[0] + ef_ref[1], e8_ref[...],
                   preferred_element_type=jnp.float32)
    h = jnp.maximum(agg, 0.0)
    mu = jnp.mean(h, axis=1, keepdims=True)
    var = jnp.mean((h - mu) ** 2, axis=1, keepdims=True)
    h = (h - mu) / jnp.sqrt(var + 1e-5) * gb_ref[0:1] + gb_ref[1:2]
    for ch in range(4):
        piece = h[:, ch * 32:(ch + 1) * 32]
        if r_ref is not None:
            piece = piece + r_ref[ch]
        out_ref[ch] = piece


def _make_tc_layer(ns, with_res):
    in_specs = [
        pl.BlockSpec((ns, BN, 32), lambda i: (0, i, 0)),
        pl.BlockSpec((ns, 32, 128), lambda i: (0, 0, 0)),
        pl.BlockSpec((2, BN, 8), lambda i: (0, i, 0)),
        pl.BlockSpec((8, 128), lambda i: (0, 0)),
        pl.BlockSpec((2, 128), lambda i: (0, 0)),
    ]
    if with_res:
        in_specs.append(pl.BlockSpec((4, BN, 32), lambda i: (0, i, 0)))
    return pl.pallas_call(
        functools.partial(_tc_layer_body, ns, with_res),
        grid=(NBLK,),
        in_specs=in_specs,
        out_specs=pl.BlockSpec((4, BN, 32), lambda i: (0, i, 0)),
        out_shape=jax.ShapeDtypeStruct((4, N_NODES, 32), jnp.float32),
    )


_tc_layer0 = _make_tc_layer(2, False)
_tc_layer1 = _make_tc_layer(4, True)
_tc_layer2 = _make_tc_layer(4, False)


def _ln_row(h, g, b):
    mu = jnp.mean(h, axis=1, keepdims=True)
    var = jnp.mean((h - mu) ** 2, axis=1, keepdims=True)
    return (h - mu) / jnp.sqrt(var + 1e-5) * g + b


def _attn_head_body(x_ref, pe_ref, wq_ref, bq_ref, wk_ref, bk_ref,
                    wv_ref, bv_ref, wo_ref, bo_ref, g_ref, hm_ref,
                    w1a_ref, w1b_ref, b1_ref, g1_ref, n1_ref,
                    w2_ref, b2_ref, g2_ref, n2_ref,
                    w3_ref, b3_ref, g3_ref, n3_ref,
                    w4_ref, b4_ref, out_ref,
                    qs, m_sc, l_sc, acc_sc):
    i = pl.program_id(0)

    @pl.when(i == 0)
    def _init():
        q = jnp.dot(pe_ref[...], wq_ref[...],
                    preferred_element_type=jnp.float32) + bq_ref[...]
        qs[...] = q / jnp.sqrt(32.0)
        m_sc[...] = jnp.full((4, 1), -1e30, jnp.float32)
        l_sc[...] = jnp.zeros((4, 1), jnp.float32)
        acc_sc[...] = jnp.zeros((4, 128), jnp.float32)

    k = jnp.dot(x_ref[0], wk_ref[0:32], preferred_element_type=jnp.float32)
    v = jnp.dot(x_ref[0], wv_ref[0:32], preferred_element_type=jnp.float32)
    for ch in range(1, 4):
        k += jnp.dot(x_ref[ch], wk_ref[ch * 32:(ch + 1) * 32],
                     preferred_element_type=jnp.float32)
        v += jnp.dot(x_ref[ch], wv_ref[ch * 32:(ch + 1) * 32],
                     preferred_element_type=jnp.float32)
    k = k + bk_ref[...]
    v = v + bv_ref[...]
    kq = k * qs[...]
    sT = lax.dot_general(g_ref[...], kq, (((0,), (1,)), ((), ())),
                         preferred_element_type=jnp.float32)
    m_blk = jnp.max(sT, axis=1, keepdims=True)
    m_new = jnp.maximum(m_sc[...], m_blk)
    alpha = jnp.exp(m_sc[...] - m_new)
    p = jnp.exp(sT - m_new)
    l_sc[...] = l_sc[...] * alpha + jnp.sum(p, axis=1, keepdims=True)
    acc_sc[...] = acc_sc[...] * alpha + lax.dot_general(
        p, v, (((1,), (0,)), ((), ())), preferred_element_type=jnp.float32)
    m_sc[...] = m_new

    @pl.when(i == NBLK - 1)
    def _final():
        att = acc_sc[...] / l_sc[...]
        att1 = jnp.sum(att * hm_ref[...], axis=0, keepdims=True)
        o = jnp.dot(att1, wo_ref[...],
                    preferred_element_type=jnp.float32) + bo_ref[...]
        h = (jnp.dot(pe_ref[...], w1a_ref[...], preferred_element_type=jnp.float32)
             + jnp.dot(o, w1b_ref[...], preferred_element_type=jnp.float32)
             + b1_ref[...])
        h = jnp.maximum(_ln_row(h, g1_ref[...], n1_ref[...]), 0.0)
        h = jnp.dot(h, w2_ref[...], preferred_element_type=jnp.float32) + b2_ref[...]
        h = jnp.maximum(_ln_row(h, g2_ref[...], n2_ref[...]), 0.0)
        h = jnp.dot(h, w3_ref[...], preferred_element_type=jnp.float32) + b3_ref[...]
        h = jnp.maximum(_ln_row(h, g3_ref[...], n3_ref[...]), 0.0)
        pred = jnp.dot(h, w4_ref[...], preferred_element_type=jnp.float32) + b4_ref[...]
        out_ref[...] = pred


def _full(shape):
    nd = len(shape)
    return pl.BlockSpec(shape, lambda i: (0,) * nd)


_attn_head = pl.pallas_call(
    _attn_head_body,
    grid=(NBLK,),
    in_specs=[
        pl.BlockSpec((4, BN, 32), lambda i: (0, i, 0)),
        _full((1, 480)), _full((480, 128)), _full((1, 128)),
        _full((128, 128)), _full((1, 128)),
        _full((128, 128)), _full((1, 128)),
        _full((128, 128)), _full((1, 128)),
        _full((128, 4)), _full((4, 128)),
        _full((480, 512)), _full((128, 512)), _full((1, 512)),
        _full((1, 512)), _full((1, 512)),
        _full((512, 256)), _full((1, 256)), _full((1, 256)), _full((1, 256)),
        _full((256, 128)), _full((1, 128)), _full((1, 128)), _full((1, 128)),
        _full((128, 1)), _full((1, 1)),
    ],
    out_specs=_full((1, 1)),
    out_shape=jax.ShapeDtypeStruct((1, 1), jnp.float32),
    scratch_shapes=[
        pltpu.VMEM((1, 128), jnp.float32),
        pltpu.VMEM((4, 1), jnp.float32),
        pltpu.VMEM((4, 1), jnp.float32),
        pltpu.VMEM((4, 128), jnp.float32),
    ],
)


def kernel(proteinEmbedding, nodeFeatures, edgeIndex, edgeFeatures, params):
    f32 = jnp.float32
    src = edgeIndex[:, 0]
    dst = edgeIndex[:, 1]
    pad = EPAD - N_EDGES
    srcp = jnp.concatenate([src, jnp.zeros((pad,), jnp.int32)]).reshape(NW, ROWS, BATCH)
    dstp = jnp.concatenate(
        [dst, jnp.full((pad,), N_NODES, jnp.int32)]).reshape(NW, ROWS, BATCH)
    efx = jnp.concatenate(
        [edgeFeatures, jnp.zeros((N_EDGES, 1), f32), jnp.ones((N_EDGES, 1), f32)], 1)
    efxp = jnp.concatenate([efx, jnp.zeros((pad, 8), f32)], 0)
    z8 = jnp.zeros((FLUSH, 8), f32)
    x0 = jnp.pad(nodeFeatures, ((0, 0), (0, 13)))

    gcn = params['gcn']
    ws = []
    e8s = []
    gbs = []
    for i, p in enumerate(gcn):
        if i == 0:
            w = jnp.pad(p['W'], ((0, 13), (0, 0)))
            ws.append(jnp.stack([w, w]))
        else:
            ws.append(p['W'].reshape(4, 32, 128))
        e8s.append(jnp.concatenate(
            [p['We'], jnp.zeros((1, 128), f32), (p['b'] + p['be'])[None]], 0))
        gbs.append(jnp.stack([p['g'], p['bn']]))

    EF = _sc_efagg(dstp, efxp, z8)
    S0 = _sc_seg1(srcp, dstp, x0)
    h0 = _tc_layer0(S0, ws[0], EF, e8s[0], gbs[0])
    S1 = _sc_seg4(srcp, dstp, h0)
    h1 = _tc_layer1(S1, ws[1], EF, e8s[1], gbs[1], h0)
    S2 = _sc_seg4(srcp, dstp, h1)
    h2 = _tc_layer2(S2, ws[2], EF, e8s[2], gbs[2])

    mlp = params['mlp']
    G = (jnp.arange(128)[:, None] // 32 == jnp.arange(4)[None, :]).astype(f32)
    HM = G.T
    pred = _attn_head(
        h2, proteinEmbedding[None], params['Wq'], params['bq'][None],
        params['Wk'], params['bk'][None], params['Wv'], params['bv'][None],
        params['Wo'], params['bo'][None], G, HM,
        mlp[0]['W'][:480], mlp[0]['W'][480:], mlp[0]['b'][None],
        mlp[0]['g'][None], mlp[0]['bn'][None],
        mlp[1]['W'], mlp[1]['b'][None], mlp[1]['g'][None], mlp[1]['bn'][None],
        mlp[2]['W'], mlp[2]['b'][None], mlp[2]['g'][None], mlp[2]['bn'][None],
        mlp[3]['W'], mlp[3]['b'][None],
    )
    return pred.reshape(1)


# speedup vs baseline: 3.8602x; 1.0042x over previous
import functools

import jax
import jax.numpy as jnp
from jax import lax
from jax.experimental import pallas as pl
from jax.experimental.pallas import tpu as pltpu
from jax.experimental.pallas import tpu_sc as plsc

N_NODES = 50000
N_EDGES = 800000

NCORE = 2
NSUB = 16
NW = NCORE * NSUB
BATCH = 128
ROWS = 200
KSUB = 40
EPAD = NW * ROWS * BATCH
ACC_ROWS = 50176
ZROWS = 56
FLUSH = 3136

_MESH = plsc.VectorSubcoreMesh(core_axis_name="c", subcore_axis_name="s")
_SC_PARAMS = pltpu.CompilerParams(use_tc_tiling_on_sc=False)


def _zero_zbuf(zbuf):
    def body(i, _):
        zbuf[i // 2, pl.ds((i % 2) * 16, 16)] = jnp.zeros((16,), jnp.float32)
        return 0
    lax.fori_loop(0, ZROWS * 2, body, 0, unroll=4)


def _zero_stripe(acc, zbuf, s):
    for z in range(FLUSH // ZROWS):
        pltpu.sync_copy(zbuf, acc.at[pl.ds(s * FLUSH + z * ZROWS, ZROWS)])


def _pipe_kblock(start_g, wait_g, start_s, wait_s):
    start_g(0, 0)
    start_g(1, 1)
    start_g(2, 2)

    def tbody(t, _):
        for q in range(4):
            j = t * 4 + q
            wait_g(j, q)
            nq = (q + 3) % 4
            if q == 0:
                start_g(j + 3, nq)
            else:
                @pl.when(t < KSUB // 4 - 1)
                def _(j=j, nq=nq):
                    start_g(j + 3, nq)
            start_s(j, q)
            wait_s(j, q)
        return 0
    lax.fori_loop(0, KSUB // 4, tbody, 0)


@functools.partial(
    pl.kernel,
    out_type=jax.ShapeDtypeStruct((4, ACC_ROWS, 32), jnp.float32),
    mesh=_MESH,
    compiler_params=_SC_PARAMS,
    scratch_types=[
        pltpu.VMEM_SHARED((ACC_ROWS, 32), jnp.float32),
        pltpu.VMEM((KSUB, BATCH), jnp.int32),
        pltpu.VMEM((KSUB, BATCH), jnp.int32),
        pltpu.VMEM((4, BATCH, 32), jnp.float32),
        pltpu.VMEM((ZROWS, 32), jnp.float32),
        [pltpu.SemaphoreType.DMA] * 4,
        [pltpu.SemaphoreType.DMA] * 4,
    ],
)
def _sc_seg4(src_hbm, dst_hbm, x_hbm, out_hbm,
             acc, srcv, dstv, rows, zbuf, gsem, ssem):
    c = lax.axis_index("c")
    s = lax.axis_index("s")
    _zero_zbuf(zbuf)
    for ci in range(2):
        chunk = c * 2 + ci
        _zero_stripe(acc, zbuf, s)
        plsc.subcore_barrier()
        for w in range(2):
            wid = s * 2 + w

            def kbody(k, _):
                pltpu.sync_copy(src_hbm.at[wid].at[pl.ds(k * KSUB, KSUB)], srcv)
                pltpu.sync_copy(dst_hbm.at[wid].at[pl.ds(k * KSUB, KSUB)], dstv)
                _pipe_kblock(
                    lambda j, p: pltpu.async_copy(
                        x_hbm.at[chunk].at[srcv.at[j]], rows.at[p], gsem[p]),
                    lambda j, p: pltpu.make_async_copy(
                        x_hbm.at[chunk].at[srcv.at[j]], rows.at[p], gsem[p]).wait(),
                    lambda j, p: pltpu.async_copy(
                        rows.at[p], acc.at[dstv.at[j]], ssem[p], add=True),
                    lambda j, p: pltpu.make_async_copy(
                        rows.at[p], acc.at[dstv.at[j]], ssem[p]).wait())
                return 0
            lax.fori_loop(0, ROWS // KSUB, kbody, 0)
        plsc.subcore_barrier()
        pltpu.sync_copy(acc.at[pl.ds(s * FLUSH, FLUSH)],
                        out_hbm.at[chunk].at[pl.ds(s * FLUSH, FLUSH)])
        plsc.subcore_barrier()


@functools.partial(
    pl.kernel,
    out_type=jax.ShapeDtypeStruct((2, ACC_ROWS, 32), jnp.float32),
    mesh=_MESH,
    compiler_params=_SC_PARAMS,
    scratch_types=[
        pltpu.VMEM_SHARED((ACC_ROWS, 32), jnp.float32),
        pltpu.VMEM((KSUB, BATCH), jnp.int32),
        pltpu.VMEM((KSUB, BATCH), jnp.int32),
        pltpu.VMEM((4, BATCH, 32), jnp.float32),
        pltpu.VMEM((ZROWS, 32), jnp.float32),
        [pltpu.SemaphoreType.DMA] * 4,
        [pltpu.SemaphoreType.DMA] * 4,
    ],
)
def _sc_seg1(src_hbm, dst_hbm, x_hbm, out_hbm,
             acc, srcv, dstv, rows, zbuf, gsem, ssem):
    c = lax.axis_index("c")
    s = lax.axis_index("s")
    _zero_zbuf(zbuf)
    _zero_stripe(acc, zbuf, s)
    plsc.subcore_barrier()
    wid = s * 2 + c

    def kbody(k, _):
        pltpu.sync_copy(src_hbm.at[wid].at[pl.ds(k * KSUB, KSUB)], srcv)
        pltpu.sync_copy(dst_hbm.at[wid].at[pl.ds(k * KSUB, KSUB)], dstv)
        _pipe_kblock(
            lambda j, p: pltpu.async_copy(
                x_hbm.at[srcv.at[j]], rows.at[p], gsem[p]),
            lambda j, p: pltpu.make_async_copy(
                x_hbm.at[srcv.at[j]], rows.at[p], gsem[p]).wait(),
            lambda j, p: pltpu.async_copy(
                rows.at[p], acc.at[dstv.at[j]], ssem[p], add=True),
            lambda j, p: pltpu.make_async_copy(
                rows.at[p], acc.at[dstv.at[j]], ssem[p]).wait())
        return 0
    lax.fori_loop(0, ROWS // KSUB, kbody, 0)
    plsc.subcore_barrier()
    pltpu.sync_copy(acc.at[pl.ds(s * FLUSH, FLUSH)],
                    out_hbm.at[c].at[pl.ds(s * FLUSH, FLUSH)])


@functools.partial(
    pl.kernel,
    out_type=jax.ShapeDtypeStruct((2, ACC_ROWS, 8), jnp.float32),
    mesh=_MESH,
    compiler_params=_SC_PARAMS,
    scratch_types=[
        pltpu.VMEM_SHARED((ACC_ROWS, 8), jnp.float32),
        pltpu.VMEM((KSUB, BATCH), jnp.int32),
        pltpu.VMEM((4, BATCH, 8), jnp.float32),
        [pltpu.SemaphoreType.DMA] * 4,
        [pltpu.SemaphoreType.DMA] * 4,
    ],
)
def _sc_efagg(dst_hbm, efx_hbm, z8_hbm, out_hbm, acc, dstv, rowbuf, gsem, ssem):
    c = lax.axis_index("c")
    s = lax.axis_index("s")
    pltpu.sync_copy(z8_hbm, acc.at[pl.ds(s * FLUSH, FLUSH)])
    plsc.subcore_barrier()
    wid = s * 2 + c
    base = wid * (ROWS * BATCH)

    def kbody(k, _):
        pltpu.sync_copy(dst_hbm.at[wid].at[pl.ds(k * KSUB, KSUB)], dstv)
        kb = base + k * KSUB * BATCH
        _pipe_kblock(
            lambda j, p: pltpu.async_copy(
                efx_hbm.at[pl.ds(kb + j * BATCH, BATCH)], rowbuf.at[p], gsem[p]),
            lambda j, p: pltpu.make_async_copy(
                efx_hbm.at[pl.ds(kb + j * BATCH, BATCH)], rowbuf.at[p], gsem[p]).wait(),
            lambda j, p: pltpu.async_copy(
                rowbuf.at[p], acc.at[dstv.at[j]], ssem[p], add=True),
            lambda j, p: pltpu.make_async_copy(
                rowbuf.at[p], acc.at[dstv.at[j]], ssem[p]).wait())
        return 0
    lax.fori_loop(0, ROWS // KSUB, kbody, 0)
    plsc.subcore_barrier()
    pltpu.sync_copy(acc.at[pl.ds(s * FLUSH, FLUSH)],
                    out_hbm.at[c].at[pl.ds(s * FLUSH, FLUSH)])


BN = 1000
NBLK = 50


def _tc_layer_body(ns, with_res, *refs):
    if with_res:
        s_ref, ws_ref, ef_ref, e8_ref, gb_ref, r_ref, out_ref = refs
    else:
        s_ref, ws_ref, ef_ref, e8_ref, gb_ref, out_ref = refs
        r_ref = None
    agg = jnp.dot(s_ref[0], ws_ref[0], preferred_element_type=jnp.float32)
    for n in range(1, ns):
        agg += jnp.dot(s_ref[n], ws_ref[n], preferred_element_type=jnp.float32)
    agg += jnp.dot(ef_ref[0] + ef_ref[1], e8_ref[...],
                   preferred_element_type=jnp.float32)
    h = jnp.maximum(agg, 0.0)
    mu = jnp.mean(h, axis=1, keepdims=True)
    var = jnp.mean((h - mu) ** 2, axis=1, keepdims=True)
    h = (h - mu) / jnp.sqrt(var + 1e-5) * gb_ref[0:1] + gb_ref[1:2]
    for ch in range(4):
        piece = h[:, ch * 32:(ch + 1) * 32]
        if r_ref is not None:
            piece = piece + r_ref[ch]
        out_ref[ch] = piece


def _make_tc_layer(ns, with_res):
    in_specs = [
        pl.BlockSpec((ns, BN, 32), lambda i: (0, i, 0)),
        pl.BlockSpec((ns, 32, 128), lambda i: (0, 0, 0)),
        pl.BlockSpec((2, BN, 8), lambda i: (0, i, 0)),
        pl.BlockSpec((8, 128), lambda i: (0, 0)),
        pl.BlockSpec((2, 128), lambda i: (0, 0)),
    ]
    if with_res:
        in_specs.append(pl.BlockSpec((4, BN, 32), lambda i: (0, i, 0)))
    return pl.pallas_call(
        functools.partial(_tc_layer_body, ns, with_res),
        grid=(NBLK,),
        in_specs=in_specs,
        out_specs=pl.BlockSpec((4, BN, 32), lambda i: (0, i, 0)),
        out_shape=jax.ShapeDtypeStruct((4, N_NODES, 32), jnp.float32),
    )


_tc_layer0 = _make_tc_layer(2, False)
_tc_layer1 = _make_tc_layer(4, True)
_tc_layer2 = _make_tc_layer(4, False)


def _ln_row(h, g, b):
    mu = jnp.mean(h, axis=1, keepdims=True)
    var = jnp.mean((h - mu) ** 2, axis=1, keepdims=True)
    return (h - mu) / jnp.sqrt(var + 1e-5) * g + b


def _attn_head_body(x_ref, pe_ref, wq_ref, bq_ref, wk_ref, bk_ref,
                    wv_ref, bv_ref, wo_ref, bo_ref, g_ref, hm_ref,
                    w1a_ref, w1b_ref, b1_ref, g1_ref, n1_ref,
                    w2_ref, b2_ref, g2_ref, n2_ref,
                    w3_ref, b3_ref, g3_ref, n3_ref,
                    w4_ref, b4_ref, out_ref,
                    qs, m_sc, l_sc, acc_sc):
    i = pl.program_id(0)

    @pl.when(i == 0)
    def _init():
        q = jnp.dot(pe_ref[...], wq_ref[...],
                    preferred_element_type=jnp.float32) + bq_ref[...]
        qs[...] = q / jnp.sqrt(32.0)
        m_sc[...] = jnp.full((4, 1), -1e30, jnp.float32)
        l_sc[...] = jnp.zeros((4, 1), jnp.float32)
        acc_sc[...] = jnp.zeros((4, 128), jnp.float32)

    k = jnp.dot(x_ref[0], wk_ref[0:32], preferred_element_type=jnp.float32)
    v = jnp.dot(x_ref[0], wv_ref[0:32], preferred_element_type=jnp.float32)
    for ch in range(1, 4):
        k += jnp.dot(x_ref[ch], wk_ref[ch * 32:(ch + 1) * 32],
                     preferred_element_type=jnp.float32)
        v += jnp.dot(x_ref[ch], wv_ref[ch * 32:(ch + 1) * 32],
                     preferred_element_type=jnp.float32)
    k = k + bk_ref[...]
    v = v + bv_ref[...]
    kq = k * qs[...]
    sT = lax.dot_general(g_ref[...], kq, (((0,), (1,)), ((), ())),
                         preferred_element_type=jnp.float32)
    m_blk = jnp.max(sT, axis=1, keepdims=True)
    m_new = jnp.maximum(m_sc[...], m_blk)
    alpha = jnp.exp(m_sc[...] - m_new)
    p = jnp.exp(sT - m_new)
    l_sc[...] = l_sc[...] * alpha + jnp.sum(p, axis=1, keepdims=True)
    acc_sc[...] = acc_sc[...] * alpha + lax.dot_general(
        p, v, (((1,), (0,)), ((), ())), preferred_element_type=jnp.float32)
    m_sc[...] = m_new

    @pl.when(i == NBLK - 1)
    def _final():
        att = acc_sc[...] / l_sc[...]
        att1 = jnp.sum(att * hm_ref[...], axis=0, keepdims=True)
        o = jnp.dot(att1, wo_ref[...],
                    preferred_element_type=jnp.float32) + bo_ref[...]
        h = (jnp.dot(pe_ref[...], w1a_ref[...], preferred_element_type=jnp.float32)
             + jnp.dot(o, w1b_ref[...], preferred_element_type=jnp.float32)
             + b1_ref[...])
        h = jnp.maximum(_ln_row(h, g1_ref[...], n1_ref[...]), 0.0)
        h = jnp.dot(h, w2_ref[...], preferred_element_type=jnp.float32) + b2_ref[...]
        h = jnp.maximum(_ln_row(h, g2_ref[...], n2_ref[...]), 0.0)
        h = jnp.dot(h, w3_ref[...], preferred_element_type=jnp.float32) + b3_ref[...]
        h = jnp.maximum(_ln_row(h, g3_ref[...], n3_ref[...]), 0.0)
        pred = jnp.dot(h, w4_ref[...], preferred_element_type=jnp.float32) + b4_ref[...]
        out_ref[...] = pred


def _full(shape):
    nd = len(shape)
    return pl.BlockSpec(shape, lambda i: (0,) * nd)


_attn_head = pl.pallas_call(
    _attn_head_body,
    grid=(NBLK,),
    in_specs=[
        pl.BlockSpec((4, BN, 32), lambda i: (0, i, 0)),
        _full((1, 480)), _full((480, 128)), _full((1, 128)),
        _full((128, 128)), _full((1, 128)),
        _full((128, 128)), _full((1, 128)),
        _full((128, 128)), _full((1, 128)),
        _full((128, 4)), _full((4, 128)),
        _full((480, 512)), _full((128, 512)), _full((1, 512)),
        _full((1, 512)), _full((1, 512)),
        _full((512, 256)), _full((1, 256)), _full((1, 256)), _full((1, 256)),
        _full((256, 128)), _full((1, 128)), _full((1, 128)), _full((1, 128)),
        _full((128, 1)), _full((1, 1)),
    ],
    out_specs=_full((1, 1)),
    out_shape=jax.ShapeDtypeStruct((1, 1), jnp.float32),
    scratch_shapes=[
        pltpu.VMEM((1, 128), jnp.float32),
        pltpu.VMEM((4, 1), jnp.float32),
        pltpu.VMEM((4, 1), jnp.float32),
        pltpu.VMEM((4, 128), jnp.float32),
    ],
)


def kernel(proteinEmbedding, nodeFeatures, edgeIndex, edgeFeatures, params):
    f32 = jnp.float32
    src = edgeIndex[:, 0]
    dst = edgeIndex[:, 1]
    pad = EPAD - N_EDGES
    srcp = jnp.concatenate([src, jnp.zeros((pad,), jnp.int32)]).reshape(NW, ROWS, BATCH)
    dstp = jnp.concatenate(
        [dst, jnp.full((pad,), N_NODES, jnp.int32)]).reshape(NW, ROWS, BATCH)
    efx = jnp.concatenate(
        [edgeFeatures, jnp.zeros((N_EDGES, 1), f32), jnp.ones((N_EDGES, 1), f32)], 1)
    efxp = jnp.concatenate([efx, jnp.zeros((pad, 8), f32)], 0)
    z8 = jnp.zeros((FLUSH, 8), f32)
    x0 = jnp.pad(nodeFeatures, ((0, 0), (0, 13)))

    gcn = params['gcn']
    ws = []
    e8s = []
    gbs = []
    for i, p in enumerate(gcn):
        if i == 0:
            w = jnp.pad(p['W'], ((0, 13), (0, 0)))
            ws.append(jnp.stack([w, w]))
        else:
            ws.append(p['W'].reshape(4, 32, 128))
        e8s.append(jnp.concatenate(
            [p['We'], jnp.zeros((1, 128), f32), (p['b'] + p['be'])[None]], 0))
        gbs.append(jnp.stack([p['g'], p['bn']]))

    EF = _sc_efagg(dstp, efxp, z8)
    S0 = _sc_seg1(srcp, dstp, x0)
    h0 = _tc_layer0(S0, ws[0], EF, e8s[0], gbs[0])
    S1 = _sc_seg4(srcp, dstp, h0)
    h1 = _tc_layer1(S1, ws[1], EF, e8s[1], gbs[1], h0)
    S2 = _sc_seg4(srcp, dstp, h1)
    h2 = _tc_layer2(S2, ws[2], EF, e8s[2], gbs[2])

    mlp = params['mlp']
    G = (jnp.arange(128)[:, None] // 32 == jnp.arange(4)[None, :]).astype(f32)
    HM = G.T
    pred = _attn_head(
        h2, proteinEmbedding[None], params['Wq'], params['bq'][None],
        params['Wk'], params['bk'][None], params['Wv'], params['bv'][None],
        params['Wo'], params['bo'][None], G, HM,
        mlp[0]['W'][:480], mlp[0]['W'][480:], mlp[0]['b'][None],
        mlp[0]['g'][None], mlp[0]['bn'][None],
        mlp[1]['W'], mlp[1]['b'][None], mlp[1]['g'][None], mlp[1]['bn'][None],
        mlp[2]['W'], mlp[2]['b'][None], mlp[2]['g'][None], mlp[2]['bn'][None],
        mlp[3]['W'], mlp[3]['b'][None],
    )
    return pred.reshape(1)
